# baseline (speedup 1.0000x reference)
import functools

import jax
import jax.numpy as jnp
from jax import lax
from jax.experimental import pallas as pl
from jax.experimental.pallas import tpu as pltpu
from jax.experimental.pallas import tpu_sc as plsc

PATCH = 1024
HEADS = 16
HPAIRS = HEADS // 2
POS_BND = int((4 * PATCH) ** (1 / 3) * 2)
RPE_NUM = 2 * POS_BND + 1

B = 8
S = 512
N = B * S * S
NC, NS, L = 2, 16, 16
NW = NC * NS
PER_W = N // NW
BP = 4096
BPR = BP // S
NBLK = PER_W // BP
NGRP = BP // L
GPR = S // L
Q_PER_B = (S * S) // PER_W
ROWS_W = PER_W // S

_T01_ROWS = RPE_NUM * RPE_NUM
_C01 = (POS_BND * RPE_NUM + POS_BND) * HPAIRS
_C2 = POS_BND * HPAIRS


def _rpe_body(t01_hbm, t2_hbm, coord_hbm, out_hbm, t01_v, t2_v, coord_v, out_v):
    wid = lax.axis_index("c") * NS + lax.axis_index("s")
    b = wid // Q_PER_B
    q = wid % Q_PER_B

    pltpu.sync_copy(t01_hbm, t01_v)
    pltpu.sync_copy(t2_hbm, t2_v)

    def block_body(blk, _):
        brow0 = q * ROWS_W + blk * BPR
        pltpu.sync_copy(coord_hbm.at[b, :, pl.ds(brow0, BPR), :], coord_v)

        @plsc.parallel_loop(0, NGRP, unroll=8)
        def group_body(g):
            r = g // GPR
            col = (g % GPR) * L
            c0 = coord_v[0, r, pl.ds(col, L)]
            c1 = coord_v[1, r, pl.ds(col, L)]
            c2 = coord_v[2, r, pl.ds(col, L)]
            c0 = jnp.minimum(jnp.maximum(c0, -POS_BND), POS_BND)
            c1 = jnp.minimum(jnp.maximum(c1, -POS_BND), POS_BND)
            c2 = jnp.minimum(jnp.maximum(c2, -POS_BND), POS_BND)
            p01 = c0 * (RPE_NUM * HPAIRS) + c1 * HPAIRS + _C01
            p2 = c2 * HPAIRS + _C2
            for k in range(HPAIRS):
                w01 = plsc.load_gather(t01_v, [p01 + k])
                w2 = plsc.load_gather(t2_v, [p2 + k])
                s = plsc.bitcast(w01, jnp.bfloat16) + plsc.bitcast(w2, jnp.bfloat16)
                lo, hi = plsc.unpack(
                    s, format=plsc.PackFormat.INTERLEAVED,
                    preferred_element_type=jnp.float32,
                )
                out_v[2 * k, r, pl.ds(col, L)] = lo
                out_v[2 * k + 1, r, pl.ds(col, L)] = hi

        pltpu.sync_copy(out_v, out_hbm.at[b, :, pl.ds(brow0, BPR), :])
        return 0

    lax.fori_loop(0, NBLK, block_body, 0)


_rpe_call = functools.partial(
    pl.kernel,
    out_type=jax.ShapeDtypeStruct((B, HEADS, S, S), jnp.float32),
    mesh=plsc.VectorSubcoreMesh(
        core_axis_name="c", subcore_axis_name="s", num_cores=NC, num_subcores=NS
    ),
    scratch_types=[
        pltpu.VMEM((_T01_ROWS * HPAIRS,), jnp.int32),
        pltpu.VMEM((RPE_NUM * HPAIRS,), jnp.int32),
        pltpu.VMEM((3, BPR, S), jnp.int32),
        pltpu.VMEM((HEADS, BPR, S), jnp.float32),
    ],
    compiler_params=pltpu.CompilerParams(
        needs_layout_passes=False, use_tc_tiling_on_sc=True
    ),
)(_rpe_body)


def _pack_pairs(t):
    tb = t.astype(jnp.bfloat16).reshape(t.shape[0], HPAIRS, 2)
    return lax.bitcast_convert_type(tb, jnp.int32).reshape(-1)


def kernel(coord, rpe_table):
    t0 = rpe_table[0:RPE_NUM]
    t1 = rpe_table[RPE_NUM : 2 * RPE_NUM]
    t2 = rpe_table[2 * RPE_NUM : 3 * RPE_NUM]
    t01 = t0[:, None, :] + t1[None, :, :]
    return _rpe_call(
        _pack_pairs(t01.reshape(_T01_ROWS, HEADS)),
        _pack_pairs(t2),
        jnp.transpose(coord, (0, 3, 1, 2)),
    )


# speedup vs baseline: 1.1493x; 1.1493x over previous
import functools

import jax
import jax.numpy as jnp
from jax import lax
from jax.experimental import pallas as pl
from jax.experimental.pallas import tpu as pltpu
from jax.experimental.pallas import tpu_sc as plsc

PATCH = 1024
HEADS = 16
HPAIRS = HEADS // 2
POS_BND = int((4 * PATCH) ** (1 / 3) * 2)
RPE_NUM = 2 * POS_BND + 1

B = 8
S = 512
N = B * S * S
NC, NS, L = 2, 16, 16
NW = NC * NS
PER_W = N // NW
BP = 4096
BPR = BP // S
NBLK = PER_W // BP
NGRP = BP // L
GPR = S // L
Q_PER_B = (S * S) // PER_W
ROWS_W = PER_W // S

_T01_ROWS = RPE_NUM * RPE_NUM
_C01 = (POS_BND * RPE_NUM + POS_BND) * HPAIRS
_C2 = POS_BND * HPAIRS


def _rpe_body(t01_hbm, t2_hbm, coord_hbm, out_hbm, t01_v, t2_v, coord_v, out_v):
    wid = lax.axis_index("c") * NS + lax.axis_index("s")
    b = wid // Q_PER_B
    q = wid % Q_PER_B

    pltpu.sync_copy(t01_hbm, t01_v)
    pltpu.sync_copy(t2_hbm, t2_v)

    def block_body(blk, _):
        brow0 = q * ROWS_W + blk * BPR
        pltpu.sync_copy(coord_hbm.at[b, :, pl.ds(brow0, BPR), :], coord_v)

        @plsc.parallel_loop(0, NGRP, unroll=2)
        def group_body(g):
            r = g // GPR
            col = (g % GPR) * L
            c0 = coord_v[0, r, pl.ds(col, L)]
            c1 = coord_v[1, r, pl.ds(col, L)]
            c2 = coord_v[2, r, pl.ds(col, L)]
            c0 = jnp.minimum(jnp.maximum(c0, -POS_BND), POS_BND)
            c1 = jnp.minimum(jnp.maximum(c1, -POS_BND), POS_BND)
            c2 = jnp.minimum(jnp.maximum(c2, -POS_BND), POS_BND)
            p01 = c0 * (RPE_NUM * HPAIRS) + c1 * HPAIRS + _C01
            p2 = c2 * HPAIRS + _C2
            for k in range(HPAIRS):
                w01 = plsc.load_gather(t01_v, [p01 + k])
                w2 = plsc.load_gather(t2_v, [p2 + k])
                s = plsc.bitcast(w01, jnp.bfloat16) + plsc.bitcast(w2, jnp.bfloat16)
                lo, hi = plsc.unpack(
                    s, format=plsc.PackFormat.INTERLEAVED,
                    preferred_element_type=jnp.float32,
                )
                out_v[2 * k, r, pl.ds(col, L)] = lo
                out_v[2 * k + 1, r, pl.ds(col, L)] = hi

        pltpu.sync_copy(out_v, out_hbm.at[b, :, pl.ds(brow0, BPR), :])
        return 0

    lax.fori_loop(0, NBLK, block_body, 0)


_rpe_call = functools.partial(
    pl.kernel,
    out_type=jax.ShapeDtypeStruct((B, HEADS, S, S), jnp.float32),
    mesh=plsc.VectorSubcoreMesh(
        core_axis_name="c", subcore_axis_name="s", num_cores=NC, num_subcores=NS
    ),
    scratch_types=[
        pltpu.VMEM((_T01_ROWS * HPAIRS,), jnp.int32),
        pltpu.VMEM((RPE_NUM * HPAIRS,), jnp.int32),
        pltpu.VMEM((3, BPR, S), jnp.int32),
        pltpu.VMEM((HEADS, BPR, S), jnp.float32),
    ],
    compiler_params=pltpu.CompilerParams(
        needs_layout_passes=False, use_tc_tiling_on_sc=True
    ),
)(_rpe_body)


def _pack_pairs(t):
    tb = t.astype(jnp.bfloat16).reshape(t.shape[0], HPAIRS, 2)
    return lax.bitcast_convert_type(tb, jnp.int32).reshape(-1)


def kernel(coord, rpe_table):
    t0 = rpe_table[0:RPE_NUM]
    t1 = rpe_table[RPE_NUM : 2 * RPE_NUM]
    t2 = rpe_table[2 * RPE_NUM : 3 * RPE_NUM]
    t01 = t0[:, None, :] + t1[None, :, :]
    return _rpe_call(
        _pack_pairs(t01.reshape(_T01_ROWS, HEADS)),
        _pack_pairs(t2),
        jnp.transpose(coord, (0, 3, 1, 2)),
    )


# speedup vs baseline: 2.5424x; 2.2121x over previous
import functools

import jax
import jax.numpy as jnp
from jax import lax
from jax.experimental import pallas as pl
from jax.experimental.pallas import tpu as pltpu
from jax.experimental.pallas import tpu_sc as plsc

PATCH = 1024
HEADS = 16
HPAIRS = HEADS // 2
POS_BND = int((4 * PATCH) ** (1 / 3) * 2)
RPE_NUM = 2 * POS_BND + 1

B = 8
S = 512
N = B * S * S
NC, NS, L = 2, 16, 16
NW = NC * NS
PER_W = N // NW
BP = 4096
BPR = BP // S
NBLK = PER_W // BP
NGRP = BP // L
GPR = S // L
Q_PER_B = (S * S) // PER_W
ROWS_W = PER_W // S

_T01_ROWS = RPE_NUM * RPE_NUM
STRIDE = HPAIRS + 1
_C01 = (POS_BND * RPE_NUM + POS_BND) * STRIDE
_C2 = POS_BND * STRIDE


def _rpe_body(
    t01_hbm, t2_hbm, coord_hbm, out_hbm,
    t01_v, t2_v, ca_v, cb_v, oa_v, ob_v,
    sem_ca, sem_cb, sem_oa, sem_ob,
):
    wid = lax.axis_index("c") * NS + lax.axis_index("s")
    b = wid // Q_PER_B
    q = wid % Q_PER_B

    def coord_src(blk):
        return coord_hbm.at[b, :, pl.ds(q * ROWS_W + blk * BPR, BPR), :]

    pltpu.async_copy(coord_src(0), ca_v, sem_ca)
    pltpu.sync_copy(t01_hbm, t01_v)
    pltpu.sync_copy(t2_hbm, t2_v)

    def do_half_block(blk, cv, ov, sem_o, hh):
        brow0 = q * ROWS_W + blk * BPR

        @pl.when(blk > 0)
        def _():
            pltpu.make_async_copy(
                ov, out_hbm.at[b, pl.ds(hh * 8, 8), pl.ds(brow0 - BPR, BPR), :], sem_o
            ).wait()

        @plsc.parallel_loop(0, NGRP, unroll=4)
        def group_body(g):
            r = g // GPR
            col = (g % GPR) * L
            c0 = cv[0, r, pl.ds(col, L)]
            c1 = cv[1, r, pl.ds(col, L)]
            c2 = cv[2, r, pl.ds(col, L)]
            c0 = jnp.minimum(jnp.maximum(c0, -POS_BND), POS_BND)
            c1 = jnp.minimum(jnp.maximum(c1, -POS_BND), POS_BND)
            c2 = jnp.minimum(jnp.maximum(c2, -POS_BND), POS_BND)
            p01 = c0 * (RPE_NUM * STRIDE) + c1 * STRIDE + _C01
            p2 = c2 * STRIDE + _C2
            for kk in range(HPAIRS // 2):
                k = hh * (HPAIRS // 2) + kk
                w01 = plsc.load_gather(t01_v, [p01 + k])
                w2 = plsc.load_gather(t2_v, [p2 + k])
                s = plsc.bitcast(w01, jnp.bfloat16) + plsc.bitcast(w2, jnp.bfloat16)
                lo, hi = plsc.unpack(
                    s, format=plsc.PackFormat.INTERLEAVED,
                    preferred_element_type=jnp.float32,
                )
                ov[2 * kk, r, pl.ds(col, L)] = lo
                ov[2 * kk + 1, r, pl.ds(col, L)] = hi

        pltpu.async_copy(
            ov, out_hbm.at[b, pl.ds(hh * 8, 8), pl.ds(brow0, BPR), :], sem_o
        )

    def pair_body(i, _):
        for half, (cv, sem_c, cv_n, sem_cn) in enumerate(
            ((ca_v, sem_ca, cb_v, sem_cb), (cb_v, sem_cb, ca_v, sem_ca))
        ):
            blk = 2 * i + half
            pltpu.make_async_copy(coord_src(blk), cv, sem_c).wait()

            @pl.when(blk + 1 < NBLK)
            def _():
                pltpu.async_copy(coord_src(blk + 1), cv_n, sem_cn)

            do_half_block(blk, cv, oa_v, sem_oa, 0)
            do_half_block(blk, cv, ob_v, sem_ob, 1)
        return 0

    lax.fori_loop(0, NBLK // 2, pair_body, 0)

    last0 = q * ROWS_W + (NBLK - 1) * BPR
    pltpu.make_async_copy(
        oa_v, out_hbm.at[b, pl.ds(0, 8), pl.ds(last0, BPR), :], sem_oa
    ).wait()
    pltpu.make_async_copy(
        ob_v, out_hbm.at[b, pl.ds(8, 8), pl.ds(last0, BPR), :], sem_ob
    ).wait()


_rpe_call = functools.partial(
    pl.kernel,
    out_type=jax.ShapeDtypeStruct((B, HEADS, S, S), jnp.float32),
    mesh=plsc.VectorSubcoreMesh(
        core_axis_name="c", subcore_axis_name="s", num_cores=NC, num_subcores=NS
    ),
    scratch_types=[
        pltpu.VMEM((_T01_ROWS * STRIDE,), jnp.int32),
        pltpu.VMEM((RPE_NUM * STRIDE,), jnp.int32),
        pltpu.VMEM((3, BPR, S), jnp.int32),
        pltpu.VMEM((3, BPR, S), jnp.int32),
        pltpu.VMEM((HEADS // 2, BPR, S), jnp.float32),
        pltpu.VMEM((HEADS // 2, BPR, S), jnp.float32),
        pltpu.SemaphoreType.DMA,
        pltpu.SemaphoreType.DMA,
        pltpu.SemaphoreType.DMA,
        pltpu.SemaphoreType.DMA,
    ],
    compiler_params=pltpu.CompilerParams(
        needs_layout_passes=False, use_tc_tiling_on_sc=True
    ),
)(_rpe_body)


def _pack_pairs(t):
    tb = t.astype(jnp.bfloat16).reshape(t.shape[0], HPAIRS, 2)
    w = lax.bitcast_convert_type(tb, jnp.int32)
    w = jnp.pad(w, ((0, 0), (0, STRIDE - HPAIRS)))
    return w.reshape(-1)


def kernel(coord, rpe_table):
    t0 = rpe_table[0:RPE_NUM]
    t1 = rpe_table[RPE_NUM : 2 * RPE_NUM]
    t2 = rpe_table[2 * RPE_NUM : 3 * RPE_NUM]
    t01 = t0[:, None, :] + t1[None, :, :]
    return _rpe_call(
        _pack_pairs(t01.reshape(_T01_ROWS, HEADS)),
        _pack_pairs(t2),
        jnp.transpose(coord, (0, 3, 1, 2)),
    )
